# Initial kernel scaffold; baseline (speedup 1.0000x reference)
import jax
import jax.numpy as jnp
from jax.experimental import pallas as pl


def kernel(users, items, edge_index, edge_vals, item_entities, item_relations, user_emb, item_emb, entity_emb, relation_emb, fc_W, fc_b):
    raise NotImplementedError("write your pallas kernel here")


# speedup vs baseline: 8.4316x; 8.4316x over previous
import functools

import jax
import jax.numpy as jnp
from jax import lax
from jax.experimental import pallas as pl
from jax.experimental.pallas import tpu as pltpu
from jax.experimental.pallas import tpu_sc as plsc

NUM_USERS = 30000
NUM_ITEMS = 20000
NUM_ENTITIES = 40000
E_PER_ITEM = 10
D = 64
HD = 32
ALPHA = 0.2

NC, NS = 2, 16
NW = NC * NS

NN = NUM_USERS + NUM_ITEMS
NNP = 50560
ROWS_PER_SUB = NNP // NS

NIP = 20480
ITEMS_PER_TILE = NIP // NW
GAT_CHUNK = 64
GAT_NCH = ITEMS_PER_TILE // GAT_CHUNK

NEP = 40448
SW = 128

EDGE_CHUNK = 128

USERS_PER_TILE = NUM_USERS // NW
U_REM = NUM_USERS - NW * USERS_PER_TILE
SEL_PER_SUB = 256
SEL_CHUNK = 64

_UNTILED = pltpu.CompilerParams(use_tc_tiling_on_sc=False)


def _mesh():
    return plsc.VectorSubcoreMesh(core_axis_name="c", subcore_axis_name="s")


def _proj_body(x_ref, w_ref, r_ref, b_ref, o_ref):
    t = jnp.dot(x_ref[...], w_ref[...], preferred_element_type=jnp.float32)
    o_ref[...] = (
        jnp.dot(t, r_ref[...], preferred_element_type=jnp.float32)
        + b_ref[0:1, :]
    )


def _tc_proj(x, w, relT, bias):
    n = x.shape[0]
    blk = 512
    return pl.pallas_call(
        _proj_body,
        grid=(n // blk,),
        in_specs=[
            pl.BlockSpec((blk, D), lambda i: (i, 0)),
            pl.BlockSpec((D, D), lambda i: (0, 0)),
            pl.BlockSpec((D, SW), lambda i: (0, 0)),
            pl.BlockSpec((8, SW), lambda i: (0, 0)),
        ],
        out_specs=pl.BlockSpec((blk, SW), lambda i: (i, 0)),
        out_shape=jax.ShapeDtypeStruct((n, SW), jnp.float32),
    )(x, w, relT, bias)


GAT_ROW_DMAS = GAT_CHUNK * E_PER_ITEM // 128


def _gat_kernel(ent_hbm, ieT_hbm, irT_hbm, sc_hbm, it_hbm, u2_hbm,
                h0_hbm,
                ie_all, ir_all, scidx_v, gidx_v, scM_v, rows_v, it_v,
                obA_v, obB_v, ust_v, sem, sem2):
    c = lax.axis_index("c")
    s = lax.axis_index("s")
    wid = s * NC + c
    lanes = lax.broadcasted_iota(jnp.int32, (16,), 0)
    npt = ITEMS_PER_TILE * E_PER_ITEM
    pltpu.sync_copy(ieT_hbm.at[pl.ds(wid * npt, npt)], ie_all)
    pltpu.sync_copy(irT_hbm.at[pl.ds(wid * npt, npt)], ir_all)
    iaoff = jnp.full((16,), NEP * SW)

    ur0 = wid * USERS_PER_TILE
    for half in range(2):
        pltpu.sync_copy(u2_hbm.at[half].at[pl.ds(ur0, USERS_PER_TILE)], ust_v)
        pltpu.sync_copy(ust_v, h0_hbm.at[pl.ds(half * NNP + ur0, USERS_PER_TILE)])

    @pl.when(wid == 0)
    def _():
        for half in range(2):
            pltpu.sync_copy(
                u2_hbm.at[half].at[pl.ds(32 * USERS_PER_TILE, U_REM)],
                ust_v.at[pl.ds(0, U_REM)])
            pltpu.sync_copy(ust_v.at[pl.ds(0, U_REM)],
                            h0_hbm.at[pl.ds(half * NNP + 32 * USERS_PER_TILE, U_REM)])

    def chunk(t, _):
        base_i = wid * ITEMS_PER_TILE + t * GAT_CHUNK
        pltpu.sync_copy(it_hbm.at[pl.ds(base_i * D, GAT_CHUNK * D)], it_v)
        bvec = jnp.full((16,), base_i)
        for e in range(E_PER_ITEM):
            for sub in range(GAT_CHUNK // 16):
                off = pl.ds(e * ITEMS_PER_TILE + t * GAT_CHUNK + sub * 16, 16)
                ei = ie_all[off]
                ri = ir_all[off]
                scidx_v[e, pl.ds(sub * 16, 16)] = ei * SW + ri
                scidx_v[e, pl.ds(GAT_CHUNK + sub * 16, 16)] = (
                    iaoff + (bvec + (sub * 16 + lanes)) * SW + ri)
                gidx_v[pl.ds(e * GAT_CHUNK + sub * 16, 16)] = ei
        descs = []
        for e in range(E_PER_ITEM):
            descs.append(pltpu.async_copy(sc_hbm.at[scidx_v.at[e]], scM_v.at[e], sem))
        for j in range(GAT_ROW_DMAS):
            descs.append(pltpu.async_copy(
                ent_hbm.at[gidx_v.at[pl.ds(j * 128, 128)]],
                rows_v.at[pl.ds(j * 128, 128)], sem2))
        for d_ in descs:
            d_.wait()
        for sub in range(GAT_CHUNK // 16):
            sl = pl.ds(sub * 16, 16)
            sl2 = pl.ds(GAT_CHUNK + sub * 16, 16)
            sc = []
            for e in range(E_PER_ITEM):
                off = pl.ds(e * ITEMS_PER_TILE + t * GAT_CHUNK + sub * 16, 16)
                sval = scM_v[e, sl] + scM_v[e, sl2]
                sval = jnp.where(sval > 0, sval, ALPHA * sval)
                sc.append(jnp.where(ie_all[off] != NUM_ENTITIES, sval, -9e15))
            m = sc[0]
            for e in range(1, E_PER_ITEM):
                m = jnp.maximum(m, sc[e])
            ex = [jnp.exp(v - m) for v in sc]
            tot = ex[0]
            for e in range(1, E_PER_ITEM):
                tot = tot + ex[e]
            inv = 1.0 / tot
            att = [v * inv for v in ex]
            for lane in range(16):
                li = sub * 16 + lane
                acc = [it_v[pl.ds(li * D + dg * 16, 16)] for dg in range(4)]
                for e in range(E_PER_ITEM):
                    w = jnp.full((16,), att[e][lane])
                    for dg in range(4):
                        acc[dg] = acc[dg] + w * rows_v[e * GAT_CHUNK + li, pl.ds(dg * 16, 16)]
                for dg in range(2):
                    obA_v[li, pl.ds(dg * 16, 16)] = acc[dg]
                    obB_v[li, pl.ds(dg * 16, 16)] = acc[dg + 2]
        pltpu.sync_copy(obA_v, h0_hbm.at[pl.ds(NUM_USERS + base_i, GAT_CHUNK)])
        pltpu.sync_copy(obB_v, h0_hbm.at[pl.ds(NNP + NUM_USERS + base_i, GAT_CHUNK)])
        return 0

    lax.fori_loop(0, GAT_NCH, chunk, 0)


def _run_gat(ent_p, ieT_flat, irT_flat, sc_tab, it_flat, u2):
    k = pl.kernel(
        _gat_kernel,
        out_type=jax.ShapeDtypeStruct((2 * NNP, HD), jnp.float32),
        mesh=_mesh(),
        compiler_params=_UNTILED,
        scratch_types=[
            pltpu.VMEM((ITEMS_PER_TILE * E_PER_ITEM,), jnp.int32),
            pltpu.VMEM((ITEMS_PER_TILE * E_PER_ITEM,), jnp.int32),
            pltpu.VMEM((E_PER_ITEM, 2 * GAT_CHUNK), jnp.int32),
            pltpu.VMEM((GAT_CHUNK * E_PER_ITEM,), jnp.int32),
            pltpu.VMEM((E_PER_ITEM, 2 * GAT_CHUNK), jnp.float32),
            pltpu.VMEM((GAT_CHUNK * E_PER_ITEM, D), jnp.float32),
            pltpu.VMEM((GAT_CHUNK * D,), jnp.float32),
            pltpu.VMEM((GAT_CHUNK, HD), jnp.float32),
            pltpu.VMEM((GAT_CHUNK, HD), jnp.float32),
            pltpu.VMEM((USERS_PER_TILE, HD), jnp.float32),
            pltpu.SemaphoreType.DMA,
            pltpu.SemaphoreType.DMA,
        ],
    )
    return k(ent_p, ieT_flat, irT_flat, sc_tab, it_flat, u2)


S_CH = 4
SB = S_CH * EDGE_CHUNK


def _spmm3g_kernel(nsuper, h0_hbm, sd_hbm, val_hbm, z_hbm, u_hbm, i_hbm,
                   h1_hbm, h2_hbm, h3_hbm, g_hbm,
                   eall, vall, di0, di1, r0, r1, acc,
                   uidx_v, iidx_v, ub0, ub1, ub2, ub3, ib0, ib1, ib2, ib3,
                   red_v, go_v,
                   sem_a, sem_b, sem_c, sem_d, sem_e, sem_f):
    for hin, hout in ((h0_hbm, h1_hbm), (h1_hbm, h2_hbm), (h2_hbm, h3_hbm)):
        _spmm_layer(nsuper, hin, sd_hbm, val_hbm, z_hbm, hout,
                    eall, vall, di0, di1, r0, r1, acc,
                    sem_a, sem_b, sem_c, sem_d, sem_e, sem_f)
        plsc.subcore_barrier()
    _gamma_body(h0_hbm, h1_hbm, h2_hbm, h3_hbm, u_hbm, i_hbm, g_hbm,
                uidx_v, iidx_v, ub0, ub1, ub2, ub3, ib0, ib1, ib2, ib3,
                red_v, go_v, sem_a)


def _spmm_layer(nsuper, h_hbm, sd_hbm, val_hbm, z_hbm, out_hbm,
                eall, vall, di0, di1, r0, r1, acc,
                sem_a, sem_b, sem_c, sem_d, sem_e, sem_f):
    c = lax.axis_index("c")
    s = lax.axis_index("s")
    coff = c * NNP
    pltpu.sync_copy(z_hbm, acc.at[pl.ds(s * ROWS_PER_SUB, ROWS_PER_SUB)])
    plsc.subcore_barrier()
    cvec = jnp.full((16,), coff)

    def scale(kk, r):
        for sub in range(EDGE_CHUNK // 16):
            vv = vall[pl.ds(kk * EDGE_CHUNK + sub * 16, 16)]
            for lane in range(16):
                e = sub * 16 + lane
                w = jnp.full((16,), vv[lane])
                for dg in range(2):
                    sl = pl.ds(dg * 16, 16)
                    r[e, sl] = r[e, sl] * w

    bufs = [(r0, di0, sem_a, sem_c), (r1, di1, sem_b, sem_d)]

    def body(u, _):
        sd_slice = sd_hbm.at[pl.ds((s * nsuper + u) * (2 * SB), 2 * SB)]
        v_slice = val_hbm.at[pl.ds((s * nsuper + u) * SB, SB)]
        pltpu.async_copy(sd_slice, eall, sem_e)
        pltpu.async_copy(v_slice, vall, sem_f)
        pltpu.make_async_copy(sd_slice, eall, sem_e).wait()
        pltpu.make_async_copy(v_slice, vall, sem_f).wait()
        for k in range(SB // 16):
            sl = pl.ds(k * 16, 16)
            eall[sl] = eall[sl] + cvec
        sc_descs = [None, None]
        pltpu.async_copy(h_hbm.at[eall.at[pl.ds(0, EDGE_CHUNK)]], r0, sem_a)
        for k in range(S_CH):
            r, di, sem_g, sem_s = bufs[k % 2]
            rn, _dn, sem_gn, sem_sn = bufs[(k + 1) % 2]
            if k + 1 < S_CH:
                if sc_descs[(k + 1) % 2] is not None:
                    sc_descs[(k + 1) % 2].wait()
                    sc_descs[(k + 1) % 2] = None
                pltpu.async_copy(
                    h_hbm.at[eall.at[pl.ds((k + 1) * EDGE_CHUNK, EDGE_CHUNK)]],
                    rn, sem_gn)
            pltpu.make_async_copy(
                h_hbm.at[eall.at[pl.ds(k * EDGE_CHUNK, EDGE_CHUNK)]], r, sem_g).wait()
            scale(k, r)
            for j in range(EDGE_CHUNK // 16):
                sl = pl.ds(j * 16, 16)
                di[sl] = eall[pl.ds(SB + k * EDGE_CHUNK + j * 16, 16)]
            sc_descs[k % 2] = pltpu.async_copy(r, acc.at[di], sem_s, add=True)
        for d_ in sc_descs:
            if d_ is not None:
                d_.wait()
        return 0

    lax.fori_loop(0, nsuper, body, 0)
    plsc.subcore_barrier()
    pltpu.sync_copy(acc.at[pl.ds(s * ROWS_PER_SUB, ROWS_PER_SUB)],
                    out_hbm.at[pl.ds(coff + s * ROWS_PER_SUB, ROWS_PER_SUB)])


_SPMM_SCRATCH = (
    [pltpu.VMEM((2 * SB,), jnp.int32),
     pltpu.VMEM((SB,), jnp.float32),
     pltpu.VMEM((EDGE_CHUNK,), jnp.int32),
     pltpu.VMEM((EDGE_CHUNK,), jnp.int32),
     pltpu.VMEM((EDGE_CHUNK, HD), jnp.float32),
     pltpu.VMEM((EDGE_CHUNK, HD), jnp.float32),
     pltpu.VMEM_SHARED((NNP, HD), jnp.float32)]
    + [pltpu.VMEM((SEL_CHUNK,), jnp.int32)] * 2
    + [pltpu.VMEM((SEL_CHUNK, HD), jnp.float32)] * 8
    + [pltpu.VMEM((32,), jnp.float32),
       pltpu.VMEM((SEL_CHUNK,), jnp.float32)]
    + [pltpu.SemaphoreType.DMA] * 6)


def _run_spmm3g(h0, sd, valp, zeros, u_nodes, i_nodes):
    ep = valp.shape[0]
    nsuper = ep // (NS * SB)
    hs = jax.ShapeDtypeStruct((2 * NNP, HD), jnp.float32)
    gs = jax.ShapeDtypeStruct((NC, u_nodes.shape[0]), jnp.float32)
    k = pl.kernel(
        functools.partial(_spmm3g_kernel, nsuper),
        out_type=(hs, hs, hs, gs),
        mesh=_mesh(),
        compiler_params=_UNTILED,
        scratch_types=_SPMM_SCRATCH,
    )
    return k(h0, sd, valp, zeros, u_nodes, i_nodes)


def _gamma_body(h0, h1, h2, h3, u_hbm, i_hbm, out_hbm,
                uidx_v, iidx_v, ub0, ub1, ub2, ub3, ib0, ib1, ib2, ib3,
                red_v, go_v, sem):
    c = lax.axis_index("c")
    s = lax.axis_index("s")
    cvec = jnp.full((16,), c * NNP)
    lanes = lax.broadcasted_iota(jnp.int32, (16,), 0)
    ubs = (ub0, ub1, ub2, ub3)
    ibs = (ib0, ib1, ib2, ib3)

    def chunk(t, _):
        b = s * SEL_PER_SUB + t * SEL_CHUNK
        pltpu.sync_copy(u_hbm.at[pl.ds(b, SEL_CHUNK)], uidx_v)
        pltpu.sync_copy(i_hbm.at[pl.ds(b, SEL_CHUNK)], iidx_v)
        for k in range(SEL_CHUNK // 16):
            sl = pl.ds(k * 16, 16)
            uidx_v[sl] = uidx_v[sl] + cvec
            iidx_v[sl] = iidx_v[sl] + cvec
        descs = []
        for h, ub, ib in zip((h0, h1, h2, h3), ubs, ibs):
            descs.append(pltpu.async_copy(h.at[uidx_v], ub, sem))
            descs.append(pltpu.async_copy(h.at[iidx_v], ib, sem))
        for d_ in descs:
            d_.wait()
        acc = jnp.zeros((16,), jnp.float32)
        for r in range(SEL_CHUNK):
            sl0, sl1 = pl.ds(0, 16), pl.ds(16, 16)
            u0 = ub0[r, sl0] + ub1[r, sl0] + ub2[r, sl0] + ub3[r, sl0]
            u1 = ub0[r, sl1] + ub1[r, sl1] + ub2[r, sl1] + ub3[r, sl1]
            i0 = ib0[r, sl0] + ib1[r, sl0] + ib2[r, sl0] + ib3[r, sl0]
            i1 = ib0[r, sl1] + ib1[r, sl1] + ib2[r, sl1] + ib3[r, sl1]
            q = u0 * i0 + u1 * i1
            for sh in (8, 4, 2, 1):
                red_v[pl.ds(0, 16)] = q
                red_v[pl.ds(16, 16)] = q
                q = q + red_v[pl.ds(sh, 16)]
            acc = jnp.where(lanes == (r % 16), q * (1.0 / 16.0), acc)
            if r % 16 == 15:
                go_v[pl.ds((r // 16) * 16, 16)] = acc
                acc = jnp.zeros((16,), jnp.float32)
        pltpu.sync_copy(go_v, out_hbm.at[c].at[pl.ds(b, SEL_CHUNK)])
        return 0

    lax.fori_loop(0, SEL_PER_SUB // SEL_CHUNK, chunk, 0)


def kernel(users, items, edge_index, edge_vals, item_entities, item_relations,
           user_emb, item_emb, entity_emb, relation_emb, fc_W, fc_b):
    f32 = jnp.float32
    i32 = jnp.int32

    relT = jnp.zeros((D, SW), f32).at[:, :relation_emb.shape[0]].set(relation_emb.T)
    brel = jnp.zeros((8, SW), f32).at[0, :relation_emb.shape[0]].set(fc_b @ relation_emb.T)
    item_emb_p = jnp.zeros((NIP, D), f32).at[:NUM_ITEMS].set(item_emb)
    ent_p = jnp.zeros((NEP, D), f32).at[:entity_emb.shape[0]].set(entity_emb)
    itemA = _tc_proj(item_emb_p, fc_W[:D], relT, brel)
    entB = _tc_proj(ent_p, fc_W[D:], relT, jnp.zeros((8, SW), f32))

    ie_p = jnp.full((NIP, E_PER_ITEM), NUM_ENTITIES, i32).at[:NUM_ITEMS].set(
        item_entities.astype(i32))
    ir_p = jnp.zeros((NIP, E_PER_ITEM), i32).at[:NUM_ITEMS].set(
        item_relations.astype(i32))
    ieTT = ie_p.reshape(NW, ITEMS_PER_TILE, E_PER_ITEM).transpose(0, 2, 1).reshape(-1)
    irTT = ir_p.reshape(NW, ITEMS_PER_TILE, E_PER_ITEM).transpose(0, 2, 1).reshape(-1)
    sc_tab = jnp.concatenate([entB.reshape(-1), itemA.reshape(-1)])
    u_halves = jnp.stack([user_emb[:, :HD], user_emb[:, HD:]]).astype(f32)
    h0 = _run_gat(ent_p, ieTT, irTT, sc_tab, item_emb_p.reshape(-1), u_halves)

    n_edges = edge_index.shape[1]
    ep = ((n_edges + NS * SB - 1) // (NS * SB)) * (NS * SB)
    pad_n = ep - n_edges
    fill = jnp.arange(pad_n, dtype=i32) % NN
    srcp = jnp.concatenate([edge_index[0].astype(i32), fill])
    dstp = jnp.concatenate([edge_index[1].astype(i32), fill])
    valp = jnp.concatenate([edge_vals.astype(f32), jnp.zeros((pad_n,), f32)])
    sd = jnp.stack([srcp.reshape(-1, SB), dstp.reshape(-1, SB)], axis=1).reshape(-1)
    zeros = jnp.zeros((ROWS_PER_SUB, HD), f32)

    u_nodes = users.astype(i32)
    i_nodes = NUM_USERS + items.astype(i32)
    _h1, _h2, _h3, gpart = _run_spmm3g(h0, sd, valp, zeros, u_nodes, i_nodes)
    return gpart[0] + gpart[1]
